# baseline (speedup 1.0000x reference)
import functools

import jax
import jax.numpy as jnp
from jax import lax
from jax.experimental import pallas as pl
from jax.experimental.pallas import tpu as pltpu
from jax.experimental.pallas import tpu_sc as plsc

N_EDGES = 800000
D = 64
V0, V1, V2 = 5, 6, 2
NCOMBO = V0 * V1 * V2

NC, NS = 2, 16
NW = NC * NS
PER_W = N_EDGES // NW
SUB = 128
CHUNK = 1024
N_FULL = PER_W // CHUNK
TAIL = PER_W - N_FULL * CHUNK
TAIL_PAD = 512


def _edge_encode_sc(attr_flat, table):
    mesh = plsc.VectorSubcoreMesh(core_axis_name="c", subcore_axis_name="s")

    @functools.partial(
        pl.kernel,
        out_type=jax.ShapeDtypeStruct((N_EDGES, D), jnp.float32),
        mesh=mesh,
        scratch_types=[
            pltpu.VMEM((3 * CHUNK,), jnp.int32),
            pltpu.VMEM((CHUNK,), jnp.int32),
            pltpu.VMEM((CHUNK, D), jnp.float32),
            pltpu.SemaphoreType.DMA,
        ],
        compiler_params=pltpu.CompilerParams(
            needs_layout_passes=False, use_tc_tiling_on_sc=False
        ),
    )
    def kern(attr_hbm, table_hbm, out_hbm, attr_v, idx_v, rows_v, sem):
        wid = lax.axis_index("s") * NC + lax.axis_index("c")
        base = wid * PER_W

        def do_chunk(start, n_real, n_sub):
            pltpu.sync_copy(
                attr_hbm.at[pl.ds(3 * start, 3 * n_real)],
                attr_v.at[pl.ds(0, 3 * n_real)],
            )

            def group(g, carry):
                e = g * 16 + lax.iota(jnp.int32, 16)
                i0 = plsc.load_gather(attr_v, [3 * e])
                i1 = plsc.load_gather(attr_v, [3 * e + 1])
                i2 = plsc.load_gather(attr_v, [3 * e + 2])
                c = (V1 * V2) * i0 + V2 * i1 + i2
                c = jnp.minimum(jnp.maximum(c, 0), NCOMBO - 1)
                idx_v[pl.ds(g * 16, 16)] = c
                return carry

            lax.fori_loop(0, (n_sub * SUB) // 16, group, 0)

            copies = [
                pltpu.async_copy(
                    table_hbm.at[idx_v.at[pl.ds(j * SUB, SUB)]],
                    rows_v.at[pl.ds(j * SUB, SUB)],
                    sem,
                )
                for j in range(n_sub)
            ]
            for cp in copies:
                cp.wait()

            pltpu.sync_copy(
                rows_v.at[pl.ds(0, n_real)],
                out_hbm.at[pl.ds(start, n_real)],
            )

        def body(ci, carry):
            do_chunk(base + ci * CHUNK, CHUNK, CHUNK // SUB)
            return carry

        lax.fori_loop(0, N_FULL, body, 0)
        do_chunk(base + N_FULL * CHUNK, TAIL, TAIL_PAD // SUB)

    return kern(attr_flat, table)


def kernel(edge_attr, W0, W1, W2):
    table = (
        W0[:, None, None, :] + W1[None, :, None, :] + W2[None, None, :, :]
    ).reshape(NCOMBO, D)
    attr_flat = edge_attr.reshape(-1).astype(jnp.int32)
    return _edge_encode_sc(attr_flat, table)


# speedup vs baseline: 1.4756x; 1.4756x over previous
import functools

import jax
import jax.numpy as jnp
from jax import lax
from jax.experimental import pallas as pl
from jax.experimental.pallas import tpu as pltpu
from jax.experimental.pallas import tpu_sc as plsc

N_EDGES = 800000
D = 64
V0, V1, V2 = 5, 6, 2
NCOMBO = V0 * V1 * V2

NC, NS = 2, 16
NW = NC * NS
PER_W = N_EDGES // NW
CHUNK = 256
N_FULL = PER_W // CHUNK
TAIL = PER_W - N_FULL * CHUNK
G_FULL = CHUNK // 16
G_TAIL = (TAIL + 15) // 16
ATTR_PAD = 3 * (PER_W + 2 * CHUNK)


def _edge_encode_sc(attr_flat, table_flat):
    mesh = plsc.VectorSubcoreMesh(core_axis_name="c", subcore_axis_name="s")

    @functools.partial(
        pl.kernel,
        out_type=jax.ShapeDtypeStruct((N_EDGES * D,), jnp.float32),
        mesh=mesh,
        scratch_types=[
            pltpu.VMEM((NCOMBO * D,), jnp.float32),
            pltpu.VMEM((ATTR_PAD,), jnp.int32),
            pltpu.VMEM((CHUNK * D,), jnp.float32),
            pltpu.VMEM((CHUNK * D,), jnp.float32),
            pltpu.SemaphoreType.DMA,
            pltpu.SemaphoreType.DMA,
        ],
        compiler_params=pltpu.CompilerParams(
            needs_layout_passes=False, use_tc_tiling_on_sc=False
        ),
    )
    def kern(attr_hbm, table_hbm, out_hbm, table_v, attr_v, rows0, rows1,
             so0, so1):
        wid = lax.axis_index("s") * NC + lax.axis_index("c")
        base = wid * PER_W

        rows = (rows0, rows1)
        sos = (so0, so1)

        pltpu.sync_copy(table_hbm, table_v)
        pltpu.sync_copy(
            attr_hbm.at[pl.ds(3 * base, 3 * PER_W)],
            attr_v.at[pl.ds(0, 3 * PER_W)],
        )

        def fire_out(ci, p, n):
            pltpu.async_copy(
                rows[p].at[pl.ds(0, n * D)],
                out_hbm.at[pl.ds((base + ci * CHUNK) * D, n * D)],
                sos[p],
            )

        def wait_out(p, n):
            pltpu.make_async_copy(
                rows[p].at[pl.ds(0, n * D)],
                out_hbm.at[pl.ds(0, n * D)],
                sos[p],
            ).wait()

        def compute(ci, p, ngroups):
            r_v = rows[p]

            def grp(g, carry):
                e_loc = g * 16 + lax.iota(jnp.int32, 16)
                t = 3 * (ci * CHUNK + e_loc)
                i0 = plsc.load_gather(attr_v, [t])
                i1 = plsc.load_gather(attr_v, [t + 1])
                i2 = plsc.load_gather(attr_v, [t + 2])
                c = (V1 * V2) * i0 + V2 * i1 + i2
                c = jnp.minimum(jnp.maximum(c, 0), NCOMBO - 1)
                cb = c * D
                eb = e_loc * D
                for col in range(D):
                    vals = plsc.load_gather(table_v, [cb + col])
                    plsc.store_scatter(r_v, [eb + col], vals)
                return carry

            lax.fori_loop(0, ngroups, grp, 0)

        compute(0, 0, G_FULL)
        fire_out(0, 0, CHUNK)
        compute(1, 1, G_FULL)
        fire_out(1, 1, CHUNK)

        def body(i2, carry):
            c0 = 2 + 2 * i2
            wait_out(0, CHUNK)
            compute(c0, 0, G_FULL)
            fire_out(c0, 0, CHUNK)
            wait_out(1, CHUNK)
            compute(c0 + 1, 1, G_FULL)
            fire_out(c0 + 1, 1, CHUNK)
            return carry

        lax.fori_loop(0, (N_FULL - 3) // 2, body, 0)

        wait_out(0, CHUNK)
        compute(N_FULL - 1, 0, G_FULL)
        fire_out(N_FULL - 1, 0, CHUNK)
        wait_out(1, CHUNK)
        compute(N_FULL, 1, G_TAIL)
        fire_out(N_FULL, 1, TAIL)
        wait_out(0, CHUNK)
        wait_out(1, TAIL)

    return kern(attr_flat, table_flat)


def kernel(edge_attr, W0, W1, W2):
    table = (
        W0[:, None, None, :] + W1[None, :, None, :] + W2[None, None, :, :]
    ).reshape(NCOMBO * D)
    attr_flat = edge_attr.reshape(-1).astype(jnp.int32)
    return _edge_encode_sc(attr_flat, table).reshape(N_EDGES, D)


# speedup vs baseline: 2.0913x; 1.4173x over previous
import functools

import jax
import jax.numpy as jnp
from jax import lax
from jax.experimental import pallas as pl
from jax.experimental.pallas import tpu as pltpu
from jax.experimental.pallas import tpu_sc as plsc

N_EDGES = 800000
D = 64
V0, V1, V2 = 5, 6, 2
NCOMBO = V0 * V1 * V2

NC, NS = 2, 16
NW = NC * NS
PER_W = N_EDGES // NW
CHUNK = 256
N_FULL = PER_W // CHUNK
TAIL = PER_W - N_FULL * CHUNK
G_FULL = CHUNK // 16
G_TAIL = (TAIL + 15) // 16
ATTR_PAD = 3 * (PER_W + 2 * CHUNK)


def _edge_encode_sc(attr_flat, table_flat):
    mesh = plsc.VectorSubcoreMesh(core_axis_name="c", subcore_axis_name="s")

    @functools.partial(
        pl.kernel,
        out_type=jax.ShapeDtypeStruct((N_EDGES * D,), jnp.float32),
        mesh=mesh,
        scratch_types=[
            pltpu.VMEM((NCOMBO * D,), jnp.float32),
            pltpu.VMEM((ATTR_PAD,), jnp.int32),
            pltpu.VMEM((CHUNK * D,), jnp.float32),
            pltpu.VMEM((CHUNK * D,), jnp.float32),
            pltpu.SemaphoreType.DMA,
            pltpu.SemaphoreType.DMA,
        ],
        compiler_params=pltpu.CompilerParams(
            needs_layout_passes=False, use_tc_tiling_on_sc=False
        ),
    )
    def kern(attr_hbm, table_hbm, out_hbm, table_v, attr_v, rows0, rows1,
             so0, so1):
        wid = lax.axis_index("s") * NC + lax.axis_index("c")
        base = wid * PER_W

        rows = (rows0, rows1)
        sos = (so0, so1)

        pltpu.sync_copy(table_hbm, table_v)
        pltpu.sync_copy(
            attr_hbm.at[pl.ds(3 * base, 3 * PER_W)],
            attr_v.at[pl.ds(0, 3 * PER_W)],
        )

        def fire_out(ci, p, n):
            pltpu.async_copy(
                rows[p].at[pl.ds(0, n * D)],
                out_hbm.at[pl.ds((base + ci * CHUNK) * D, n * D)],
                sos[p],
            )

        def wait_out(p, n):
            pltpu.make_async_copy(
                rows[p].at[pl.ds(0, n * D)],
                out_hbm.at[pl.ds(0, n * D)],
                sos[p],
            ).wait()

        def compute(ci, p, ngroups):
            r_v = rows[p]

            def grp(g, carry):
                e_loc = g * 16 + lax.iota(jnp.int32, 16)
                t = 3 * (ci * CHUNK + e_loc)
                i0 = plsc.load_gather(attr_v, [t])
                i1 = plsc.load_gather(attr_v, [t + 1])
                i2 = plsc.load_gather(attr_v, [t + 2])
                c = (V1 * V2) * i0 + V2 * i1 + i2
                c = jnp.minimum(jnp.maximum(c, 0), NCOMBO - 1)
                cb = c * D
                rb0 = g * (16 * D)
                for e in range(16):
                    cbe = cb[e]
                    rb = rb0 + e * D
                    for k in range(D // 16):
                        r_v[pl.ds(rb + 16 * k, 16)] = table_v[
                            pl.ds(cbe + 16 * k, 16)
                        ]
                return carry

            lax.fori_loop(0, ngroups, grp, 0)

        compute(0, 0, G_FULL)
        fire_out(0, 0, CHUNK)
        compute(1, 1, G_FULL)
        fire_out(1, 1, CHUNK)

        def body(i2, carry):
            c0 = 2 + 2 * i2
            wait_out(0, CHUNK)
            compute(c0, 0, G_FULL)
            fire_out(c0, 0, CHUNK)
            wait_out(1, CHUNK)
            compute(c0 + 1, 1, G_FULL)
            fire_out(c0 + 1, 1, CHUNK)
            return carry

        lax.fori_loop(0, (N_FULL - 3) // 2, body, 0)

        wait_out(0, CHUNK)
        compute(N_FULL - 1, 0, G_FULL)
        fire_out(N_FULL - 1, 0, CHUNK)
        wait_out(1, CHUNK)
        compute(N_FULL, 1, G_TAIL)
        fire_out(N_FULL, 1, TAIL)
        wait_out(0, CHUNK)
        wait_out(1, TAIL)

    return kern(attr_flat, table_flat)


def kernel(edge_attr, W0, W1, W2):
    table = (
        W0[:, None, None, :] + W1[None, :, None, :] + W2[None, None, :, :]
    ).reshape(NCOMBO * D)
    attr_flat = edge_attr.reshape(-1).astype(jnp.int32)
    return _edge_encode_sc(attr_flat, table).reshape(N_EDGES, D)


# speedup vs baseline: 2.4247x; 1.1594x over previous
import functools

import jax
import jax.numpy as jnp
from jax import lax
from jax.experimental import pallas as pl
from jax.experimental.pallas import tpu as pltpu
from jax.experimental.pallas import tpu_sc as plsc

N_EDGES = 800000
D = 64
V0, V1, V2 = 5, 6, 2
NCOMBO = V0 * V1 * V2

NC, NS = 2, 16
NW = NC * NS
G = 128
NGROUPS = N_EDGES // G
G_BASE = NGROUPS // NW
G_EXTRA = NGROUPS - G_BASE * NW
MAX_NG = G_BASE + 1


def _edge_encode_sc(attr_flat, table_t):
    mesh = plsc.VectorSubcoreMesh(core_axis_name="c", subcore_axis_name="s")

    @functools.partial(
        pl.kernel,
        out_type=jax.ShapeDtypeStruct((D, N_EDGES), jnp.float32),
        mesh=mesh,
        scratch_types=[
            pltpu.VMEM((NCOMBO * D,), jnp.float32),
            pltpu.VMEM((3 * MAX_NG * G,), jnp.int32),
            pltpu.VMEM((D // 8, 8, G), jnp.float32),
            pltpu.VMEM((D // 8, 8, G), jnp.float32),
            pltpu.SemaphoreType.DMA,
            pltpu.SemaphoreType.DMA,
        ],
        compiler_params=pltpu.CompilerParams(
            needs_layout_passes=False, use_tc_tiling_on_sc=True
        ),
    )
    def kern(attr_hbm, table_hbm, out_hbm, table_v, attr_v, tiles0, tiles1,
             so0, so1):
        wid = lax.axis_index("s") * NC + lax.axis_index("c")
        g0 = wid * G_BASE + jnp.minimum(wid, G_EXTRA)
        ng = G_BASE + jnp.where(wid < G_EXTRA, 1, 0)

        tiles = (tiles0, tiles1)
        sos = (so0, so1)

        pltpu.sync_copy(table_hbm, table_v)
        pltpu.sync_copy(
            attr_hbm.at[pl.ds(3 * g0 * G, 3 * G_BASE * G)],
            attr_v.at[pl.ds(0, 3 * G_BASE * G)],
        )

        @pl.when(ng > G_BASE)
        def _extra_attr():
            pltpu.sync_copy(
                attr_hbm.at[pl.ds(3 * (g0 + G_BASE) * G, 3 * G)],
                attr_v.at[pl.ds(3 * G_BASE * G, 3 * G)],
            )

        def fire_out(g, p):
            for r8 in range(D // 8):
                pltpu.async_copy(
                    tiles[p].at[r8],
                    out_hbm.at[pl.ds(8 * r8, 8), pl.ds((g0 + g) * G, G)],
                    sos[p],
                )

        def wait_out(p):
            for r8 in range(D // 8):
                pltpu.make_async_copy(
                    tiles[p].at[r8],
                    out_hbm.at[pl.ds(8 * r8, 8), pl.ds(0, G)],
                    sos[p],
                ).wait()

        def compute(g, p):
            t_v = tiles[p]

            def sub(l, carry):
                e = g * G + l * 16 + lax.iota(jnp.int32, 16)
                t = 3 * e
                i0 = plsc.load_gather(attr_v, [t])
                i1 = plsc.load_gather(attr_v, [t + 1])
                i2 = plsc.load_gather(attr_v, [t + 2])
                c = (V1 * V2) * i0 + V2 * i1 + i2
                c = jnp.minimum(jnp.maximum(c, 0), NCOMBO - 1)
                for col in range(D):
                    v = plsc.load_gather(table_v, [c + col * NCOMBO])
                    t_v[col // 8, col % 8, pl.ds(l * 16, 16)] = v
                return carry

            lax.fori_loop(0, G // 16, sub, 0)

        compute(0, 0)
        fire_out(0, 0)
        compute(1, 1)
        fire_out(1, 1)

        def body(i2, carry):
            g = 2 + 2 * i2
            wait_out(0)
            compute(g, 0)
            fire_out(g, 0)
            wait_out(1)
            compute(g + 1, 1)
            fire_out(g + 1, 1)
            return carry

        lax.fori_loop(0, (ng - 2) // 2, body, 0)

        @pl.when(ng % 2 == 1)
        def _odd_tail():
            wait_out(0)
            compute(ng - 1, 0)
            fire_out(ng - 1, 0)

        wait_out(0)
        wait_out(1)

    return kern(attr_flat, table_t)


def kernel(edge_attr, W0, W1, W2):
    table = (
        W0[:, None, None, :] + W1[None, :, None, :] + W2[None, None, :, :]
    ).reshape(NCOMBO, D)
    table_t = table.T.reshape(NCOMBO * D)
    attr_flat = edge_attr.reshape(-1).astype(jnp.int32)
    out_t = _edge_encode_sc(attr_flat, table_t)
    return out_t.T


# speedup vs baseline: 17.1802x; 7.0856x over previous
import functools

import jax
import jax.numpy as jnp
from jax import lax
from jax.experimental import pallas as pl
from jax.experimental.pallas import tpu as pltpu
from jax.experimental.pallas import tpu_sc as plsc

N_EDGES = 800000
D = 64
V0, V1, V2 = 5, 6, 2
NCOMBO = V0 * V1 * V2

NC, NS = 2, 16
NW = NC * NS
G = 128
NGROUPS = N_EDGES // G
C_GRP = 5
SUP = C_GRP * G
NSUP = NGROUPS // C_GRP
S_BASE = NSUP // NW
S_EXTRA = NSUP - S_BASE * NW


def _edge_encode_sc(c_all, table_t):
    mesh = plsc.VectorSubcoreMesh(core_axis_name="c", subcore_axis_name="s")

    @functools.partial(
        pl.kernel,
        out_type=jax.ShapeDtypeStruct((D // 8, NGROUPS, 8, G), jnp.float32),
        mesh=mesh,
        scratch_types=[
            pltpu.VMEM((NCOMBO * D,), jnp.float32),
            pltpu.VMEM((SUP,), jnp.int32),
            pltpu.VMEM((SUP,), jnp.int32),
            pltpu.VMEM((D // 8, C_GRP, 8, G), jnp.float32),
            pltpu.VMEM((D // 8, C_GRP, 8, G), jnp.float32),
            pltpu.SemaphoreType.DMA,
            pltpu.SemaphoreType.DMA,
            pltpu.SemaphoreType.DMA,
            pltpu.SemaphoreType.DMA,
        ],
        compiler_params=pltpu.CompilerParams(
            needs_layout_passes=False, use_tc_tiling_on_sc=True
        ),
    )
    def kern(c_hbm, table_hbm, out_hbm, table_v, c0, c1, tiles0, tiles1,
             sa0, sa1, so0, so1):
        wid = lax.axis_index("s") * NC + lax.axis_index("c")
        s0 = wid * S_BASE + jnp.minimum(wid, S_EXTRA)
        ns = S_BASE + jnp.where(wid < S_EXTRA, 1, 0)

        cbufs = (c0, c1)
        tiles = (tiles0, tiles1)
        sas = (sa0, sa1)
        sos = (so0, so1)

        pltpu.sync_copy(table_hbm, table_v)

        def fire_attr(s, p):
            pltpu.async_copy(
                c_hbm.at[pl.ds((s0 + s) * SUP, SUP)], cbufs[p], sas[p]
            )

        def wait_attr(p):
            pltpu.make_async_copy(
                c_hbm.at[pl.ds(0, SUP)], cbufs[p], sas[p]
            ).wait()

        def fire_out(s, p):
            pltpu.async_copy(
                tiles[p],
                out_hbm.at[:, pl.ds((s0 + s) * C_GRP, C_GRP)],
                sos[p],
            )

        def wait_out(p):
            pltpu.make_async_copy(
                tiles[p],
                out_hbm.at[:, pl.ds(0, C_GRP)],
                sos[p],
            ).wait()

        def compute(p):
            c_v = cbufs[p]
            t_v = tiles[p]

            def sub(l, carry):
                c = c_v[pl.ds(l * 16, 16)]
                c = jnp.minimum(jnp.maximum(c, 0), NCOMBO - 1)
                grp = l // 8
                si = (l % 8) * 16
                for col in range(D):
                    v = plsc.load_gather(table_v, [c + col * NCOMBO])
                    t_v[col // 8, grp, col % 8, pl.ds(si, 16)] = v
                return carry

            lax.fori_loop(0, SUP // 16, sub, 0)

        fire_attr(0, 0)
        fire_attr(1, 1)
        wait_attr(0)
        compute(0)
        fire_out(0, 0)
        fire_attr(2, 0)
        wait_attr(1)
        compute(1)
        fire_out(1, 1)
        fire_attr(3, 1)

        def body(i2, carry):
            s = 2 + 2 * i2
            wait_attr(0)
            wait_out(0)
            compute(0)
            fire_out(s, 0)

            @pl.when(s + 2 < ns)
            def _pf0():
                fire_attr(s + 2, 0)

            wait_attr(1)
            wait_out(1)
            compute(1)
            fire_out(s + 1, 1)

            @pl.when(s + 3 < ns)
            def _pf1():
                fire_attr(s + 3, 1)

            return carry

        lax.fori_loop(0, (ns - 2) // 2, body, 0)

        @pl.when(ns % 2 == 1)
        def _odd_tail():
            wait_attr(0)
            wait_out(0)
            compute(0)
            fire_out(ns - 1, 0)

        wait_out(0)
        wait_out(1)

    return kern(c_all, table_t)


def kernel(edge_attr, W0, W1, W2):
    table = (
        W0[:, None, None, :] + W1[None, :, None, :] + W2[None, None, :, :]
    ).reshape(NCOMBO, D)
    table_t = table.T.reshape(NCOMBO * D)
    a = edge_attr.astype(jnp.int32)
    c_all = (V1 * V2) * a[:, 0] + V2 * a[:, 1] + a[:, 2]
    out4 = _edge_encode_sc(c_all, table_t)
    return out4.transpose(1, 3, 0, 2).reshape(N_EDGES, D)
